# baseline (speedup 1.0000x reference)
import functools

import jax
import jax.numpy as jnp
from jax import lax
from jax.experimental import pallas as pl
from jax.experimental.pallas import tpu as pltpu
from jax.experimental.pallas import tpu_sc as plsc

_NC = 2
_NS = 16
_NW = _NC * _NS
_LANES = 16
_CHUNK = 100


def _pooled_mean_sc(idx3, emb_table, batch, hidden):
    chunks_w = idx3.shape[1]
    rows_w = chunks_w // 2
    n_col = hidden // _LANES
    inv_seq = jnp.float32(1.0 / (2 * _CHUNK))
    mesh = plsc.VectorSubcoreMesh(core_axis_name="c", subcore_axis_name="s")

    @functools.partial(
        pl.kernel,
        mesh=mesh,
        out_type=jax.ShapeDtypeStruct((batch, hidden), jnp.float32),
        compiler_params=pltpu.CompilerParams(use_tc_tiling_on_sc=False),
        scratch_types=[
            pltpu.VMEM((chunks_w, _CHUNK), jnp.int32),
            pltpu.VMEM((_CHUNK, hidden), jnp.float32),
            pltpu.VMEM((_CHUNK, hidden), jnp.float32),
            pltpu.VMEM((rows_w, hidden), jnp.float32),
            pltpu.SemaphoreType.DMA,
            pltpu.SemaphoreType.DMA,
        ],
    )
    def k(idx_hbm, table_hbm, out_hbm, idx_v, buf0, buf1, pooled_v, sem0, sem1):
        wid = lax.axis_index("s") * _NC + lax.axis_index("c")
        pltpu.sync_copy(idx_hbm.at[wid], idx_v)
        pltpu.async_copy(table_hbm.at[idx_v.at[0]], buf0, sem0)
        pltpu.async_copy(table_hbm.at[idx_v.at[1]], buf1, sem1)

        def reduce_chunk(buf, accs):
            def body(s, a):
                return tuple(
                    a[c] + buf[s, pl.ds(c * _LANES, _LANES)] for c in range(n_col)
                )

            return lax.fori_loop(0, _CHUNK, body, accs)

        def row_body(g, carry):
            c0 = 2 * g
            zeros = tuple(
                jnp.zeros((_LANES,), jnp.float32) for _ in range(n_col)
            )
            pltpu.make_async_copy(table_hbm.at[idx_v.at[c0]], buf0, sem0).wait()
            accs = reduce_chunk(buf0, zeros)

            @pl.when(g + 1 < rows_w)
            def _():
                pltpu.async_copy(table_hbm.at[idx_v.at[c0 + 2]], buf0, sem0)

            pltpu.make_async_copy(
                table_hbm.at[idx_v.at[c0 + 1]], buf1, sem1
            ).wait()
            accs = reduce_chunk(buf1, accs)

            @pl.when(g + 1 < rows_w)
            def _():
                pltpu.async_copy(table_hbm.at[idx_v.at[c0 + 3]], buf1, sem1)

            for c in range(n_col):
                pooled_v[g, pl.ds(c * _LANES, _LANES)] = accs[c] * inv_seq
            return carry

        lax.fori_loop(0, rows_w, row_body, 0)
        pltpu.sync_copy(pooled_v, out_hbm.at[pl.ds(wid * rows_w, rows_w)])

    return k(idx3, emb_table)


def _mlp_tc(pooled, W1, b1, W2, b2):
    def body(p_ref, w1_ref, b1_ref, w2_ref, b2_ref, o_ref):
        h = jnp.dot(p_ref[...], w1_ref[...], preferred_element_type=jnp.float32)
        h = jnp.maximum(h + b1_ref[...], 0.0)
        o_ref[...] = (
            jnp.dot(h, w2_ref[...], preferred_element_type=jnp.float32)
            + b2_ref[...]
        )

    return pl.pallas_call(
        body,
        out_shape=jax.ShapeDtypeStruct((pooled.shape[0], 1), jnp.float32),
    )(pooled, W1, b1, W2, b2)


def kernel(input_ids, emb_table, W1, b1, W2, b2):
    batch, seq = input_ids.shape
    hidden = emb_table.shape[1]
    chunks_w = batch * seq // (_NW * _CHUNK)
    idx3 = input_ids.astype(jnp.int32).reshape(_NW, chunks_w, _CHUNK)
    pooled = _pooled_mean_sc(idx3, emb_table, batch, hidden)
    out = _mlp_tc(
        pooled,
        W1,
        b1.reshape(1, -1).astype(jnp.float32),
        W2,
        b2.reshape(1, 1).astype(jnp.float32),
    )
    return out.reshape(batch)


# speedup vs baseline: 1.1095x; 1.1095x over previous
import functools

import jax
import jax.numpy as jnp
from jax import lax
from jax.experimental import pallas as pl
from jax.experimental.pallas import tpu as pltpu
from jax.experimental.pallas import tpu_sc as plsc

_NC = 2
_NS = 16
_NW = _NC * _NS
_LANES = 16
_CHUNK = 100
_NBUF = 4
_UNROLL = 4


def _pooled_mean_sc(idx3, emb_table, batch, hidden):
    chunks_w = idx3.shape[1]
    rows_w = chunks_w // 2
    n_col = hidden // _LANES
    inv_seq = jnp.float32(1.0 / (2 * _CHUNK))
    mesh = plsc.VectorSubcoreMesh(core_axis_name="c", subcore_axis_name="s")

    @functools.partial(
        pl.kernel,
        mesh=mesh,
        out_type=jax.ShapeDtypeStruct((batch, hidden), jnp.float32),
        compiler_params=pltpu.CompilerParams(use_tc_tiling_on_sc=False),
        scratch_types=[
            pltpu.VMEM((chunks_w, _CHUNK), jnp.int32),
            [pltpu.VMEM((_CHUNK, hidden), jnp.float32) for _ in range(_NBUF)],
            pltpu.VMEM((rows_w, hidden), jnp.float32),
            [pltpu.SemaphoreType.DMA for _ in range(_NBUF)],
        ],
    )
    def k(idx_hbm, table_hbm, out_hbm, idx_v, bufs, pooled_v, sems):
        wid = lax.axis_index("s") * _NC + lax.axis_index("c")
        pltpu.sync_copy(idx_hbm.at[wid], idx_v)
        for b in range(_NBUF):
            pltpu.async_copy(table_hbm.at[idx_v.at[b]], bufs[b], sems[b])

        def reduce_chunk(buf, accs):
            assert _CHUNK % _UNROLL == 0

            def body(i, a):
                s = i * _UNROLL
                for u in range(_UNROLL):
                    a = tuple(
                        a[c] + buf[s + u, pl.ds(c * _LANES, _LANES)]
                        for c in range(n_col)
                    )
                return a

            return lax.fori_loop(0, _CHUNK // _UNROLL, body, accs)

        def group_body(g, carry):
            c0 = _NBUF * g
            for half in range(_NBUF // 2):
                row = (_NBUF // 2) * g + half
                accs = tuple(
                    jnp.zeros((_LANES,), jnp.float32) for _ in range(n_col)
                )
                for k2 in range(2):
                    b = 2 * half + k2
                    pltpu.make_async_copy(
                        table_hbm.at[idx_v.at[c0 + b]], bufs[b], sems[b]
                    ).wait()
                    accs = reduce_chunk(bufs[b], accs)

                    @pl.when(c0 + b + _NBUF < chunks_w)
                    def _():
                        pltpu.async_copy(
                            table_hbm.at[idx_v.at[c0 + b + _NBUF]],
                            bufs[b],
                            sems[b],
                        )

                for c in range(n_col):
                    pooled_v[row, pl.ds(c * _LANES, _LANES)] = accs[c] * inv_seq
            return carry

        lax.fori_loop(0, chunks_w // _NBUF, group_body, 0)
        pltpu.sync_copy(pooled_v, out_hbm.at[pl.ds(wid * rows_w, rows_w)])

    return k(idx3, emb_table)


def _mlp_tc(pooled, W1, b1, W2, b2):
    def body(p_ref, w1_ref, b1_ref, w2_ref, b2_ref, o_ref):
        h = jnp.dot(p_ref[...], w1_ref[...], preferred_element_type=jnp.float32)
        h = jnp.maximum(h + b1_ref[...], 0.0)
        o_ref[...] = (
            jnp.dot(h, w2_ref[...], preferred_element_type=jnp.float32)
            + b2_ref[...]
        )

    return pl.pallas_call(
        body,
        out_shape=jax.ShapeDtypeStruct((pooled.shape[0], 1), jnp.float32),
    )(pooled, W1, b1, W2, b2)


def kernel(input_ids, emb_table, W1, b1, W2, b2):
    batch, seq = input_ids.shape
    hidden = emb_table.shape[1]
    chunks_w = batch * seq // (_NW * _CHUNK)
    idx3 = input_ids.astype(jnp.int32).reshape(_NW, chunks_w, _CHUNK)
    pooled = _pooled_mean_sc(idx3, emb_table, batch, hidden)
    out = _mlp_tc(
        pooled,
        W1,
        b1.reshape(1, -1).astype(jnp.float32),
        W2,
        b2.reshape(1, 1).astype(jnp.float32),
    )
    return out.reshape(batch)
